# Initial kernel scaffold; baseline (speedup 1.0000x reference)
import jax
import jax.numpy as jnp
from jax.experimental import pallas as pl


def kernel(mem, idx, val):
    raise NotImplementedError("write your pallas kernel here")


# speedup vs baseline: 4.5941x; 4.5941x over previous
import functools

import jax
import jax.numpy as jnp
from jax import lax
from jax.experimental import pallas as pl
from jax.experimental.pallas import tpu as pltpu
from jax.experimental.pallas import tpu_sc as plsc

M = 1000000
D = 128
B = 16384
DECAY_F = 0.95

NC = 2
NS = 16
L = 16
NW = NC * NS
CHUNK = 128
ROWS = B // CHUNK
H = B // NC
TRASH = H

_mesh = plsc.VectorSubcoreMesh(core_axis_name="c", subcore_axis_name="s")


def _wid():
    return lax.axis_index("s") * NC + lax.axis_index("c")


@functools.partial(
    pl.kernel,
    out_type=jax.ShapeDtypeStruct((M,), jnp.int32),
    mesh=_mesh,
    scratch_types=[
        pltpu.VMEM((CHUNK,), jnp.int32),
        pltpu.VMEM((CHUNK,), jnp.int32),
    ],
)
def _tag_kernel(idx_hbm, tag_hbm, idxb, jb):
    wid = _wid()
    n_per = ROWS // NW

    for q in range(n_per):
        row = wid * n_per + q
        pltpu.sync_copy(idx_hbm.at[row], idxb)
        j0 = row * CHUNK
        for l in range(0, CHUNK, L):
            jb.at[pl.ds(l, L)][...] = j0 + l + lax.iota(jnp.int32, L)
        pltpu.sync_copy(jb, tag_hbm.at[idxb])


@functools.partial(
    pl.kernel,
    out_type=jax.ShapeDtypeStruct((B, D), jnp.float32),
    mesh=_mesh,
    scratch_types=[
        pltpu.VMEM((CHUNK, D), jnp.float32),
        pltpu.VMEM((CHUNK,), jnp.int32),
        pltpu.VMEM((CHUNK,), jnp.int32),
        pltpu.VMEM((CHUNK,), jnp.int32),
        pltpu.VMEM_SHARED((H + 8, D), jnp.float32),
    ],
)
def _acc_kernel(idx_hbm, tag_hbm, val_hbm, acc_hbm, vbuf, idxb, wb, tb, acc_sh):
    c = lax.axis_index("c")
    s = lax.axis_index("s")
    half0 = c * H

    zrows = H // NS

    @pl.loop(0, CHUNK)
    def _(r):
        for l in range(0, D, L):
            vbuf.at[r, pl.ds(l, L)][...] = jnp.zeros((L,), jnp.float32)

    for q in range(zrows // CHUNK):
        pltpu.sync_copy(vbuf, acc_sh.at[pl.ds(s * zrows + q * CHUNK, CHUNK)])
    plsc.subcore_barrier()

    n_per = ROWS // NS

    for q in range(n_per):
        row = s * n_per + q
        pltpu.sync_copy(idx_hbm.at[row], idxb)
        pltpu.sync_copy(tag_hbm.at[idxb], wb)
        pltpu.sync_copy(val_hbm.at[pl.ds(row * CHUNK, CHUNK)], vbuf)
        for l in range(0, CHUNK, L):
            wv = wb.at[pl.ds(l, L)][...]
            local = wv - half0
            mine = (local >= 0) & (local < H)
            tb.at[pl.ds(l, L)][...] = jnp.where(mine, local, TRASH)
        pltpu.sync_copy(vbuf, acc_sh.at[tb], add=True)
    plsc.subcore_barrier()

    for q in range(zrows // CHUNK):
        r0 = s * zrows + q * CHUNK
        pltpu.sync_copy(acc_sh.at[pl.ds(r0, CHUNK)], vbuf)
        pltpu.sync_copy(vbuf, acc_hbm.at[pl.ds(half0 + r0, CHUNK)])


@functools.partial(
    pl.kernel,
    out_type=jax.ShapeDtypeStruct((B, D), jnp.float32),
    mesh=_mesh,
    scratch_types=[
        pltpu.VMEM((CHUNK,), jnp.int32),
        pltpu.VMEM((CHUNK,), jnp.int32),
        pltpu.VMEM((CHUNK, D), jnp.float32),
        pltpu.VMEM((CHUNK, D), jnp.float32),
    ],
)
def _combine_kernel(idx_hbm, tag_hbm, mem_hbm, acc_hbm, out_hbm,
                    idxb, wb, mbuf, abuf):
    wid = _wid()
    n_per = ROWS // NW

    for q in range(n_per):
        row = wid * n_per + q
        pltpu.sync_copy(idx_hbm.at[row], idxb)
        pltpu.sync_copy(tag_hbm.at[idxb], wb)
        pltpu.sync_copy(mem_hbm.at[idxb], mbuf)
        pltpu.sync_copy(acc_hbm.at[wb], abuf)

        @pl.loop(0, CHUNK)
        def _(r):
            for l in range(0, D, L):
                sl = (r, pl.ds(l, L))
                mbuf.at[sl][...] = mbuf.at[sl][...] + abuf.at[sl][...] * DECAY_F

        pltpu.sync_copy(mbuf, out_hbm.at[pl.ds(row * CHUNK, CHUNK)])


def kernel(mem, idx, val):
    idx2 = jnp.reshape(idx.astype(jnp.int32), (ROWS, CHUNK))
    tags = _tag_kernel(idx2)
    acc = _acc_kernel(idx2, tags, val)
    return _combine_kernel(idx2, tags, mem, acc)


# speedup vs baseline: 6.2168x; 1.3532x over previous
import functools

import jax
import jax.numpy as jnp
from jax import lax
from jax.experimental import pallas as pl
from jax.experimental.pallas import tpu as pltpu
from jax.experimental.pallas import tpu_sc as plsc

M = 1000000
D = 128
B = 16384
DECAY_F = 0.95

NC = 2
NS = 16
L = 16
NW = NC * NS
CHUNK = 128
ROWS = B // CHUNK
H = B // NC
TRASH = H

_mesh = plsc.VectorSubcoreMesh(core_axis_name="c", subcore_axis_name="s")


def _wid():
    return lax.axis_index("s") * NC + lax.axis_index("c")


@functools.partial(
    pl.kernel,
    out_type=jax.ShapeDtypeStruct((M,), jnp.int32),
    mesh=_mesh,
    scratch_types=[
        pltpu.VMEM((ROWS // NW, CHUNK), jnp.int32),
        pltpu.VMEM((ROWS // NW, CHUNK), jnp.int32),
        pltpu.SemaphoreType.DMA,
        pltpu.SemaphoreType.DMA,
    ],
)
def _tag_kernel(idx_hbm, tag_hbm, idxb, jb, sem_in, sem_sc):
    wid = _wid()
    n_per = ROWS // NW
    row0 = wid * n_per

    cp = pltpu.async_copy(idx_hbm.at[pl.ds(row0, n_per)], idxb, sem_in)
    for q in range(n_per):
        j0 = (row0 + q) * CHUNK
        for l in range(0, CHUNK, L):
            jb.at[q, pl.ds(l, L)][...] = j0 + l + lax.iota(jnp.int32, L)
    cp.wait()
    cps = [
        pltpu.async_copy(jb.at[q], tag_hbm.at[idxb.at[q]], sem_sc)
        for q in range(n_per)
    ]
    for cp in cps:
        cp.wait()


@functools.partial(
    pl.kernel,
    out_type=jax.ShapeDtypeStruct((B, D), jnp.float32),
    mesh=_mesh,
    scratch_types=[
        pltpu.VMEM((CHUNK, D), jnp.float32),
        pltpu.VMEM((CHUNK, D), jnp.float32),
        pltpu.VMEM((CHUNK, D), jnp.float32),
        pltpu.VMEM((ROWS // NS, CHUNK), jnp.int32),
        pltpu.VMEM((ROWS // NS, CHUNK), jnp.int32),
        pltpu.VMEM((CHUNK,), jnp.int32),
        pltpu.VMEM((CHUNK,), jnp.int32),
        pltpu.VMEM((CHUNK,), jnp.int32),
        pltpu.VMEM_SHARED((H + 8, D), jnp.float32),
        pltpu.SemaphoreType.DMA,
        pltpu.SemaphoreType.DMA,
        pltpu.SemaphoreType.DMA,
        pltpu.SemaphoreType.DMA,
    ],
)
def _acc_kernel(idx_hbm, tag_hbm, val_hbm, acc_hbm,
                vb0, vb1, vb2, idxb, wb, tb0, tb1, tb2, acc_sh,
                sem_in, sem_tag, sem_z, sem_add):
    c = lax.axis_index("c")
    s = lax.axis_index("s")
    half0 = c * H
    n_per = ROWS // NS
    row0 = s * n_per
    zrows = H // NS

    cp_idx = pltpu.async_copy(idx_hbm.at[pl.ds(row0, n_per)], idxb, sem_in)

    @pl.loop(0, CHUNK)
    def _(r):
        for l in range(0, D, L):
            vb0.at[r, pl.ds(l, L)][...] = jnp.zeros((L,), jnp.float32)

    zero_cps = [
        pltpu.async_copy(vb0, acc_sh.at[pl.ds(s * zrows + q * CHUNK, CHUNK)],
                         sem_z)
        for q in range(zrows // CHUNK)
    ]
    cp_idx.wait()
    tag_cps = [
        pltpu.async_copy(tag_hbm.at[idxb.at[q]], wb.at[q], sem_tag)
        for q in range(n_per)
    ]
    for cp in zero_cps:
        cp.wait()
    for cp in tag_cps:
        cp.wait()
    plsc.subcore_barrier()

    vbufs = (vb0, vb1, vb2)
    tbufs = (tb0, tb1, tb2)
    add_cps = [None] * n_per
    val_cps = [None] * n_per
    val_cps[0] = pltpu.async_copy(
        val_hbm.at[pl.ds(row0 * CHUNK, CHUNK)], vb0, sem_in)
    for q in range(n_per):
        vb, tb = vbufs[q % 3], tbufs[q % 3]
        if q >= 2:
            add_cps[q - 2].wait()
        if q + 1 < n_per:
            val_cps[q + 1] = pltpu.async_copy(
                val_hbm.at[pl.ds((row0 + q + 1) * CHUNK, CHUNK)],
                vbufs[(q + 1) % 3], sem_in)
        for l in range(0, CHUNK, L):
            wv = wb.at[q, pl.ds(l, L)][...]
            local = wv - half0
            mine = (local >= 0) & (local < H)
            tb.at[pl.ds(l, L)][...] = jnp.where(mine, local, TRASH)
        val_cps[q].wait()
        add_cps[q] = pltpu.async_copy(vb, acc_sh.at[tb], sem_add, add=True)
    for q in range(max(0, n_per - 2), n_per):
        add_cps[q].wait()
    plsc.subcore_barrier()

    n_dump = zrows // CHUNK
    in_cps = [None] * n_dump
    out_cps = [None] * n_dump
    in_cps[0] = pltpu.async_copy(acc_sh.at[pl.ds(s * zrows, CHUNK)], vb0,
                                 sem_in)
    in_cps[1] = pltpu.async_copy(acc_sh.at[pl.ds(s * zrows + CHUNK, CHUNK)],
                                 vb1, sem_in)
    for q in range(n_dump):
        if q + 2 < n_dump:
            if q >= 1:
                out_cps[q - 1].wait()
            in_cps[q + 2] = pltpu.async_copy(
                acc_sh.at[pl.ds(s * zrows + (q + 2) * CHUNK, CHUNK)],
                vbufs[(q + 2) % 3], sem_in)
        in_cps[q].wait()
        out_cps[q] = pltpu.async_copy(
            vbufs[q % 3],
            acc_hbm.at[pl.ds(half0 + s * zrows + q * CHUNK, CHUNK)], sem_z)
    for q in range(max(0, n_dump - 3), n_dump):
        out_cps[q].wait()


@functools.partial(
    pl.kernel,
    out_type=jax.ShapeDtypeStruct((B, D), jnp.float32),
    mesh=_mesh,
    scratch_types=[
        pltpu.VMEM((ROWS // NW, CHUNK), jnp.int32),
        pltpu.VMEM((ROWS // NW, CHUNK), jnp.int32),
        pltpu.VMEM((CHUNK, D), jnp.float32),
        pltpu.VMEM((CHUNK, D), jnp.float32),
        pltpu.VMEM((CHUNK, D), jnp.float32),
        pltpu.VMEM((CHUNK, D), jnp.float32),
        pltpu.SemaphoreType.DMA,
        pltpu.SemaphoreType.DMA,
        pltpu.SemaphoreType.DMA,
    ],
)
def _combine_kernel(idx_hbm, tag_hbm, mem_hbm, acc_hbm, out_hbm,
                    idxb, wb, mb0, mb1, ab0, ab1, sem_in, sem_g, sem_out):
    wid = _wid()
    n_per = ROWS // NW
    row0 = wid * n_per

    pltpu.sync_copy(idx_hbm.at[pl.ds(row0, n_per)], idxb)
    tag_cps = [
        pltpu.async_copy(tag_hbm.at[idxb.at[q]], wb.at[q], sem_in)
        for q in range(n_per)
    ]
    for cp in tag_cps:
        cp.wait()

    mbufs = (mb0, mb1)
    abufs = (ab0, ab1)

    def fire(q):
        b = q % 2
        return (pltpu.async_copy(mem_hbm.at[idxb.at[q]], mbufs[b], sem_g),
                pltpu.async_copy(acc_hbm.at[wb.at[q]], abufs[b], sem_g))

    g_cps = [None] * n_per
    out_cps = [None] * n_per
    g_cps[0] = fire(0)
    for q in range(n_per):
        mb, ab = mbufs[q % 2], abufs[q % 2]
        if q + 1 < n_per:
            if q >= 1:
                out_cps[q - 1].wait()
            g_cps[q + 1] = fire(q + 1)
        for cp in g_cps[q]:
            cp.wait()

        @pl.loop(0, CHUNK)
        def _(r):
            for l in range(0, D, L):
                sl = (r, pl.ds(l, L))
                mb.at[sl][...] = mb.at[sl][...] + ab.at[sl][...] * DECAY_F

        out_cps[q] = pltpu.async_copy(
            mb, out_hbm.at[pl.ds((row0 + q) * CHUNK, CHUNK)], sem_out)
    out_cps[n_per - 2].wait()
    out_cps[n_per - 1].wait()


def kernel(mem, idx, val):
    idx2 = jnp.reshape(idx.astype(jnp.int32), (ROWS, CHUNK))
    tags = _tag_kernel(idx2)
    acc = _acc_kernel(idx2, tags, val)
    return _combine_kernel(idx2, tags, mem, acc)
